# baseline (speedup 1.0000x reference)
import dataclasses

import jax
import jax.numpy as jnp
from jax import lax
from jax.experimental import pallas as pl
from jax.experimental.pallas import tpu as pltpu
from jax.experimental.pallas import tpu_sc as plsc

_SC_PARAMS = pltpu.CompilerParams()
if "needs_layout_passes" in pltpu.CompilerParams.__dataclass_fields__:
    _SC_PARAMS = dataclasses.replace(_SC_PARAMS, needs_layout_passes=False)

_NPIX = 262144
_NF = 128
_NSP = 10000
_NSPP = 10240
_NE = 60000
_NEP = 61440
_NCLS = 21
_NCORES = 2
_NSUB = 16
_NW = _NCORES * _NSUB
_PIXW = _NPIX // _NW
_CHUNK = 64
_NCHUNK = _PIXW // _CHUNK
_CNTW = 16
_NSB = 4
_SBCH = _NCHUNK // _NSB
_ROWS_W = _NSPP // _NSUB
_EW = _NEP // _NW
_ECHUNK = 64
_NECH = _EW // _ECHUNK
_EPAD = 32


def _seg_body(feat_hbm, sp_hbm, sums_hbm, cnts_hbm,
              idx_v, buf_v, hist_v, tab_sh,
              sem0, sem1):
    cid = lax.axis_index("c")
    sid = lax.axis_index("s")
    wid = cid * _NSUB + sid

    z16 = jnp.zeros((16,), jnp.float32)
    o16 = jnp.ones((16,), jnp.float32)

    @pl.loop(0, _CHUNK)
    def _(i):
        for k in range(_NF // 16):
            buf_v[0, i, pl.ds(k * 16, 16)] = z16

    @pl.loop(0, _NSPP // 16)
    def _(i):
        hist_v[pl.ds(i * 16, 16)] = z16

    base_r = sid * _ROWS_W
    for r in range(_ROWS_W // _CHUNK):
        pltpu.sync_copy(buf_v.at[0], tab_sh.at[pl.ds(base_r + r * _CHUNK, _CHUNK)])

    plsc.subcore_barrier()

    pix0 = wid * _PIXW

    def rows(c):
        return feat_hbm.at[pl.ds(pix0 + c * _CHUNK, _CHUNK)]

    for sb in range(_NSB):
        ch0 = sb * _SBCH
        pltpu.sync_copy(sp_hbm.at[pl.ds(wid * _NCHUNK + ch0, _SBCH)], idx_v)
        pltpu.async_copy(rows(ch0), buf_v.at[0], sem0)
        pltpu.async_copy(rows(ch0 + 1), buf_v.at[1], sem1)

        @pl.loop(0, _SBCH, step=2)
        def _(c):
            g = ch0 + c
            pltpu.make_async_copy(rows(g), buf_v.at[0], sem0).wait()
            pltpu.sync_copy(buf_v.at[0], tab_sh.at[idx_v.at[c]], add=True)
            for k in range(_CHUNK // 16):
                plsc.addupdate_scatter(hist_v, [idx_v[c, pl.ds(k * 16, 16)]], o16)

            @pl.when(c + 2 < _SBCH)
            def _():
                pltpu.async_copy(rows(g + 2), buf_v.at[0], sem0)

            pltpu.make_async_copy(rows(g + 1), buf_v.at[1], sem1).wait()
            pltpu.sync_copy(buf_v.at[1], tab_sh.at[idx_v.at[c + 1]], add=True)
            for k in range(_CHUNK // 16):
                plsc.addupdate_scatter(hist_v, [idx_v[c + 1, pl.ds(k * 16, 16)]], o16)

            @pl.when(c + 3 < _SBCH)
            def _():
                pltpu.async_copy(rows(g + 3), buf_v.at[1], sem1)

    plsc.subcore_barrier()

    pltpu.sync_copy(tab_sh.at[pl.ds(base_r, _ROWS_W)],
                    sums_hbm.at[cid].at[pl.ds(base_r, _ROWS_W)])
    pltpu.sync_copy(hist_v, cnts_hbm.at[wid])


def _phase1(feat, sp2d):
    f = pl.kernel(
        _seg_body,
        out_type=(jax.ShapeDtypeStruct((_NCORES, _NSPP, _NF), jnp.float32),
                  jax.ShapeDtypeStruct((_NW, _NSPP), jnp.float32)),
        mesh=plsc.VectorSubcoreMesh(core_axis_name="c", subcore_axis_name="s"),
        compiler_params=_SC_PARAMS,
        scratch_types=[
            pltpu.VMEM((_SBCH, _CHUNK), jnp.int32),
            pltpu.VMEM((2, _CHUNK, _NF), jnp.float32),
            pltpu.VMEM((_NSPP,), jnp.float32),
            pltpu.VMEM_SHARED((_NSPP, _NF), jnp.float32),
            pltpu.SemaphoreType.DMA,
            pltpu.SemaphoreType.DMA,
        ],
    )
    return f(feat, sp2d)


def _proj_body(ps_ref, pc_ref, wn_ref, we_ref, npot_ref, tab_ref):
    sums = ps_ref[0] + ps_ref[1]
    cnts = jnp.sum(pc_ref[...], axis=0)
    inv = 1.0 / jnp.clip(cnts, 1e-12, None)
    nf = sums * inv[:, None]

    def dot_t(a, b):
        return lax.dot_general(a, b, (((1,), (1,)), ((), ())),
                               preferred_element_type=jnp.float32,
                               precision=lax.Precision.HIGHEST)

    npot_ref[...] = dot_t(nf, wn_ref[...])
    zpad = jnp.zeros((64 - _NCLS, _NF), jnp.float32)
    wab = jnp.concatenate([we_ref[:, :_NF], zpad, we_ref[:, _NF:], zpad], axis=0)
    tab_ref[...] = dot_t(nf, wab)


_BLK = 1280


def _phase2(psums, pcnts, W_node, W_edge):
    nblk = _NSPP // _BLK
    return pl.pallas_call(
        _proj_body,
        grid=(nblk,),
        in_specs=[
            pl.BlockSpec((2, _BLK, _NF), lambda i: (0, i, 0)),
            pl.BlockSpec((_NW, _BLK), lambda i: (0, i)),
            pl.BlockSpec((_NCLS, _NF), lambda i: (0, 0)),
            pl.BlockSpec((_NCLS, 2 * _NF), lambda i: (0, 0)),
        ],
        out_specs=[
            pl.BlockSpec((_BLK, _NCLS), lambda i: (i, 0)),
            pl.BlockSpec((_BLK, _NF), lambda i: (i, 0)),
        ],
        out_shape=(jax.ShapeDtypeStruct((_NSP, _NCLS), jnp.float32),
                   jax.ShapeDtypeStruct((_NSPP, _NF), jnp.float32)),
    )(psums, pcnts, W_node, W_edge)


_EBLK = 1200


def _pack_body(ep_ref, out_ref):
    out_ref[...] = ep_ref[:, :_NCLS]


def _phase4(ep):
    return pl.pallas_call(
        _pack_body,
        grid=(_NE // _EBLK,),
        in_specs=[pl.BlockSpec((_EBLK, _EPAD), lambda i: (i, 0))],
        out_specs=pl.BlockSpec((_EBLK, _NCLS), lambda i: (i, 0)),
        out_shape=jax.ShapeDtypeStruct((_NE, _NCLS), jnp.float32),
    )(ep)


def _edge_body(tab_hbm, i0_hbm, i1_hbm, out_hbm,
               i0_v, i1_v, ba_v, bb_v, out_v, tab_sh, sga, sgb):
    cid = lax.axis_index("c")
    sid = lax.axis_index("s")
    wid = cid * _NSUB + sid

    seg = _NSPP // _NSUB
    pltpu.sync_copy(tab_hbm.at[pl.ds(sid * seg, seg)],
                    tab_sh.at[pl.ds(sid * seg, seg)])
    pltpu.sync_copy(i0_hbm.at[wid], i0_v)
    pltpu.sync_copy(i1_hbm.at[wid], i1_v)
    plsc.subcore_barrier()
    ebase = wid * _EW

    @pl.loop(0, _NECH)
    def _(j):
        ca = pltpu.async_copy(tab_sh.at[i0_v.at[j]], ba_v, sga)
        cb = pltpu.async_copy(tab_sh.at[i1_v.at[j]], bb_v, sgb)
        ca.wait()
        cb.wait()

        @pl.loop(0, _ECHUNK)
        def _(i):
            out_v[i, pl.ds(0, 16)] = ba_v[i, pl.ds(0, 16)] + bb_v[i, pl.ds(64, 16)]
            out_v[i, pl.ds(16, 16)] = ba_v[i, pl.ds(16, 16)] + bb_v[i, pl.ds(80, 16)]

        pltpu.sync_copy(out_v, out_hbm.at[pl.ds(ebase + j * _ECHUNK, _ECHUNK)])


def _phase3(tab, e0, e1):
    f = pl.kernel(
        _edge_body,
        out_type=jax.ShapeDtypeStruct((_NEP, _EPAD), jnp.float32),
        mesh=plsc.VectorSubcoreMesh(core_axis_name="c", subcore_axis_name="s"),
        compiler_params=_SC_PARAMS,
        scratch_types=[
            pltpu.VMEM((_NECH, _ECHUNK), jnp.int32),
            pltpu.VMEM((_NECH, _ECHUNK), jnp.int32),
            pltpu.VMEM((_ECHUNK, _NF), jnp.float32),
            pltpu.VMEM((_ECHUNK, _NF), jnp.float32),
            pltpu.VMEM((_ECHUNK, _EPAD), jnp.float32),
            pltpu.VMEM_SHARED((_NSPP, _NF), jnp.float32),
            pltpu.SemaphoreType.DMA,
            pltpu.SemaphoreType.DMA,
        ],
    )
    return f(tab, e0, e1)


def kernel(image, superpixel, edge_indexes, W_node, W_edge):
    feat = image.reshape(_NPIX, _NF)
    sp2d = superpixel.reshape(_NPIX // _CHUNK, _CHUNK)
    epad = jnp.zeros((_NEP - _NE,), jnp.int32)
    e0 = jnp.concatenate([edge_indexes[:, 0], epad]).reshape(_NW, _NECH, _ECHUNK)
    e1 = jnp.concatenate([edge_indexes[:, 1], epad]).reshape(_NW, _NECH, _ECHUNK)

    psums, pcnts = _phase1(feat, sp2d)
    npot, tab = _phase2(psums, pcnts, W_node, W_edge)
    ep = _phase3(tab, e0, e1)
    return (npot, _phase4(ep))


# speedup vs baseline: 1.0657x; 1.0657x over previous
import dataclasses

import jax
import jax.numpy as jnp
from jax import lax
from jax.experimental import pallas as pl
from jax.experimental.pallas import tpu as pltpu
from jax.experimental.pallas import tpu_sc as plsc

_SC_PARAMS = pltpu.CompilerParams()
if "needs_layout_passes" in pltpu.CompilerParams.__dataclass_fields__:
    _SC_PARAMS = dataclasses.replace(_SC_PARAMS, needs_layout_passes=False)

_NPIX = 262144
_NF = 128
_NSP = 10000
_NSPP = 10240
_NE = 60000
_NEP = 61440
_NCLS = 21
_NCORES = 2
_NSUB = 16
_NW = _NCORES * _NSUB
_PIXW = _NPIX // _NW
_CHUNK = 64
_NCHUNK = _PIXW // _CHUNK
_CNTW = 16
_NSB = 4
_SBCH = _NCHUNK // _NSB
_ROWS_W = _NSPP // _NSUB
_EW = _NEP // _NW
_ECHUNK = 64
_NECH = _EW // _ECHUNK
_EPAD = 32


def _seg_body(feat_hbm, sp_hbm, sums_hbm, cnts_hbm,
              idx_v, buf_v, hist_v, tab_sh,
              sem0, sem1):
    cid = lax.axis_index("c")
    sid = lax.axis_index("s")
    wid = cid * _NSUB + sid

    z16 = jnp.zeros((16,), jnp.float32)
    o16 = jnp.ones((16,), jnp.float32)

    @pl.loop(0, _CHUNK)
    def _(i):
        for k in range(_NF // 16):
            buf_v[0, i, pl.ds(k * 16, 16)] = z16

    @pl.loop(0, _NSPP // 16)
    def _(i):
        hist_v[pl.ds(i * 16, 16)] = z16

    base_r = sid * _ROWS_W
    for r in range(_ROWS_W // _CHUNK):
        pltpu.sync_copy(buf_v.at[0], tab_sh.at[pl.ds(base_r + r * _CHUNK, _CHUNK)])

    plsc.subcore_barrier()

    pix0 = wid * _PIXW

    def rows(c):
        return feat_hbm.at[pl.ds(pix0 + c * _CHUNK, _CHUNK)]

    for sb in range(_NSB):
        ch0 = sb * _SBCH
        pltpu.sync_copy(sp_hbm.at[pl.ds(wid * _NCHUNK + ch0, _SBCH)], idx_v)
        pltpu.async_copy(rows(ch0), buf_v.at[0], sem0)
        pltpu.async_copy(rows(ch0 + 1), buf_v.at[1], sem1)

        @pl.loop(0, _SBCH, step=2)
        def _(c):
            g = ch0 + c
            pltpu.make_async_copy(rows(g), buf_v.at[0], sem0).wait()
            pltpu.sync_copy(buf_v.at[0], tab_sh.at[idx_v.at[c]], add=True)
            for k in range(_CHUNK // 16):
                plsc.addupdate_scatter(hist_v, [idx_v[c, pl.ds(k * 16, 16)]], o16)

            @pl.when(c + 2 < _SBCH)
            def _():
                pltpu.async_copy(rows(g + 2), buf_v.at[0], sem0)

            pltpu.make_async_copy(rows(g + 1), buf_v.at[1], sem1).wait()
            pltpu.sync_copy(buf_v.at[1], tab_sh.at[idx_v.at[c + 1]], add=True)
            for k in range(_CHUNK // 16):
                plsc.addupdate_scatter(hist_v, [idx_v[c + 1, pl.ds(k * 16, 16)]], o16)

            @pl.when(c + 3 < _SBCH)
            def _():
                pltpu.async_copy(rows(g + 3), buf_v.at[1], sem1)

    plsc.subcore_barrier()

    pltpu.sync_copy(tab_sh.at[pl.ds(base_r, _ROWS_W)],
                    sums_hbm.at[cid].at[pl.ds(base_r, _ROWS_W)])
    pltpu.sync_copy(hist_v, cnts_hbm.at[wid])


def _phase1(feat, sp2d):
    f = pl.kernel(
        _seg_body,
        out_type=(jax.ShapeDtypeStruct((_NCORES, _NSPP, _NF), jnp.float32),
                  jax.ShapeDtypeStruct((_NW, _NSPP), jnp.float32)),
        mesh=plsc.VectorSubcoreMesh(core_axis_name="c", subcore_axis_name="s"),
        compiler_params=_SC_PARAMS,
        scratch_types=[
            pltpu.VMEM((_SBCH, _CHUNK), jnp.int32),
            pltpu.VMEM((2, _CHUNK, _NF), jnp.float32),
            pltpu.VMEM((_NSPP,), jnp.float32),
            pltpu.VMEM_SHARED((_NSPP, _NF), jnp.float32),
            pltpu.SemaphoreType.DMA,
            pltpu.SemaphoreType.DMA,
        ],
    )
    return f(feat, sp2d)


def _proj_body(ps_ref, pc_ref, wn_ref, we_ref, npot_ref, tab_ref):
    sums = ps_ref[0] + ps_ref[1]
    cnts = jnp.sum(pc_ref[...], axis=0)
    inv = 1.0 / jnp.clip(cnts, 1e-12, None)
    nf = sums * inv[:, None]

    def dot_t(a, b):
        return lax.dot_general(a, b, (((1,), (1,)), ((), ())),
                               preferred_element_type=jnp.float32,
                               precision=lax.Precision.HIGHEST)

    npot_ref[...] = dot_t(nf, wn_ref[...])
    zpad = jnp.zeros((64 - _NCLS, _NF), jnp.float32)
    wab = jnp.concatenate([we_ref[:, :_NF], zpad, we_ref[:, _NF:], zpad], axis=0)
    tab_ref[...] = dot_t(nf, wab)


_BLK = 1280


def _phase2(psums, pcnts, W_node, W_edge):
    nblk = _NSPP // _BLK
    return pl.pallas_call(
        _proj_body,
        grid=(nblk,),
        in_specs=[
            pl.BlockSpec((2, _BLK, _NF), lambda i: (0, i, 0)),
            pl.BlockSpec((_NW, _BLK), lambda i: (0, i)),
            pl.BlockSpec((_NCLS, _NF), lambda i: (0, 0)),
            pl.BlockSpec((_NCLS, 2 * _NF), lambda i: (0, 0)),
        ],
        out_specs=[
            pl.BlockSpec((_BLK, _NCLS), lambda i: (i, 0)),
            pl.BlockSpec((_BLK, _NF), lambda i: (i, 0)),
        ],
        out_shape=(jax.ShapeDtypeStruct((_NSP, _NCLS), jnp.float32),
                   jax.ShapeDtypeStruct((_NSPP, _NF), jnp.float32)),
    )(psums, pcnts, W_node, W_edge)


def _edge_body(tab_hbm, i0_hbm, i1_hbm, out_hbm,
               i0_v, i1_v, ba_v, bb_v, out_v, tab_sh, sga, sgb):
    cid = lax.axis_index("c")
    sid = lax.axis_index("s")
    wid = cid * _NSUB + sid

    seg = _NSPP // _NSUB
    pltpu.sync_copy(tab_hbm.at[pl.ds(sid * seg, seg)],
                    tab_sh.at[pl.ds(sid * seg, seg)])
    pltpu.sync_copy(i0_hbm.at[wid], i0_v)
    pltpu.sync_copy(i1_hbm.at[wid], i1_v)
    plsc.subcore_barrier()
    ebase = wid * _EW

    @pl.loop(0, _NECH)
    def _(j):
        ca = pltpu.async_copy(tab_sh.at[i0_v.at[j]], ba_v, sga)
        cb = pltpu.async_copy(tab_sh.at[i1_v.at[j]], bb_v, sgb)
        ca.wait()
        cb.wait()

        @pl.loop(0, _ECHUNK)
        def _(i):
            out_v[i, pl.ds(0, 16)] = ba_v[i, pl.ds(0, 16)] + bb_v[i, pl.ds(64, 16)]
            out_v[i, pl.ds(5, 16)] = ba_v[i, pl.ds(5, 16)] + bb_v[i, pl.ds(69, 16)]

        pltpu.sync_copy(out_v, out_hbm.at[pl.ds(ebase + j * _ECHUNK, _ECHUNK)])


def _phase3(tab, e0, e1):
    f = pl.kernel(
        _edge_body,
        out_type=jax.ShapeDtypeStruct((_NEP, _NCLS), jnp.float32),
        mesh=plsc.VectorSubcoreMesh(core_axis_name="c", subcore_axis_name="s"),
        compiler_params=_SC_PARAMS,
        scratch_types=[
            pltpu.VMEM((_NECH, _ECHUNK), jnp.int32),
            pltpu.VMEM((_NECH, _ECHUNK), jnp.int32),
            pltpu.VMEM((_ECHUNK, _NF), jnp.float32),
            pltpu.VMEM((_ECHUNK, _NF), jnp.float32),
            pltpu.VMEM((_ECHUNK, _NCLS), jnp.float32),
            pltpu.VMEM_SHARED((_NSPP, _NF), jnp.float32),
            pltpu.SemaphoreType.DMA,
            pltpu.SemaphoreType.DMA,
        ],
    )
    return f(tab, e0, e1)


def kernel(image, superpixel, edge_indexes, W_node, W_edge):
    feat = image.reshape(_NPIX, _NF)
    sp2d = superpixel.reshape(_NPIX // _CHUNK, _CHUNK)
    epad = jnp.zeros((_NEP - _NE,), jnp.int32)
    e0 = jnp.concatenate([edge_indexes[:, 0], epad]).reshape(_NW, _NECH, _ECHUNK)
    e1 = jnp.concatenate([edge_indexes[:, 1], epad]).reshape(_NW, _NECH, _ECHUNK)

    psums, pcnts = _phase1(feat, sp2d)
    npot, tab = _phase2(psums, pcnts, W_node, W_edge)
    ep = _phase3(tab, e0, e1)
    return (npot, ep[:_NE])


# speedup vs baseline: 1.1747x; 1.1023x over previous
import dataclasses

import jax
import jax.numpy as jnp
from jax import lax
from jax.experimental import pallas as pl
from jax.experimental.pallas import tpu as pltpu
from jax.experimental.pallas import tpu_sc as plsc

_SC_PARAMS = pltpu.CompilerParams()
if "needs_layout_passes" in pltpu.CompilerParams.__dataclass_fields__:
    _SC_PARAMS = dataclasses.replace(_SC_PARAMS, needs_layout_passes=False)

_NPIX = 262144
_NF = 128
_NSP = 10000
_NSPP = 10240
_NE = 60000
_NEP = 61440
_NCLS = 21
_NCORES = 2
_NSUB = 16
_NW = _NCORES * _NSUB
_PIXW = _NPIX // _NW
_CHUNK = 64
_NCHUNK = _PIXW // _CHUNK
_CNTW = 16
_NSB = 4
_SBCH = _NCHUNK // _NSB
_ROWS_W = _NSPP // _NSUB
_NEW = 30
_EW = _NE // _NEW
_ECHUNK = 80
_NECH = _EW // _ECHUNK
_EPAD = 32


def _seg_body(feat_hbm, sp_hbm, sums_hbm, cnts_hbm,
              idx_v, buf_v, hist_v, tab_sh,
              sem0, sem1):
    cid = lax.axis_index("c")
    sid = lax.axis_index("s")
    wid = cid * _NSUB + sid

    z16 = jnp.zeros((16,), jnp.float32)
    o16 = jnp.ones((16,), jnp.float32)

    @pl.loop(0, _CHUNK)
    def _(i):
        for k in range(_NF // 16):
            buf_v[0, i, pl.ds(k * 16, 16)] = z16

    @pl.loop(0, _NSPP // 16)
    def _(i):
        hist_v[pl.ds(i * 16, 16)] = z16

    base_r = sid * _ROWS_W
    for r in range(_ROWS_W // _CHUNK):
        pltpu.sync_copy(buf_v.at[0], tab_sh.at[pl.ds(base_r + r * _CHUNK, _CHUNK)])

    plsc.subcore_barrier()

    pix0 = wid * _PIXW

    def rows(c):
        return feat_hbm.at[pl.ds(pix0 + c * _CHUNK, _CHUNK)]

    for sb in range(_NSB):
        ch0 = sb * _SBCH
        pltpu.sync_copy(sp_hbm.at[pl.ds(wid * _NCHUNK + ch0, _SBCH)], idx_v)
        pltpu.async_copy(rows(ch0), buf_v.at[0], sem0)
        pltpu.async_copy(rows(ch0 + 1), buf_v.at[1], sem1)

        @pl.loop(0, _SBCH, step=2)
        def _(c):
            g = ch0 + c
            pltpu.make_async_copy(rows(g), buf_v.at[0], sem0).wait()
            pltpu.sync_copy(buf_v.at[0], tab_sh.at[idx_v.at[c]], add=True)
            for k in range(_CHUNK // 16):
                plsc.addupdate_scatter(hist_v, [idx_v[c, pl.ds(k * 16, 16)]], o16)

            @pl.when(c + 2 < _SBCH)
            def _():
                pltpu.async_copy(rows(g + 2), buf_v.at[0], sem0)

            pltpu.make_async_copy(rows(g + 1), buf_v.at[1], sem1).wait()
            pltpu.sync_copy(buf_v.at[1], tab_sh.at[idx_v.at[c + 1]], add=True)
            for k in range(_CHUNK // 16):
                plsc.addupdate_scatter(hist_v, [idx_v[c + 1, pl.ds(k * 16, 16)]], o16)

            @pl.when(c + 3 < _SBCH)
            def _():
                pltpu.async_copy(rows(g + 3), buf_v.at[1], sem1)

    plsc.subcore_barrier()

    pltpu.sync_copy(tab_sh.at[pl.ds(base_r, _ROWS_W)],
                    sums_hbm.at[cid].at[pl.ds(base_r, _ROWS_W)])
    pltpu.sync_copy(hist_v, cnts_hbm.at[wid])


def _phase1(feat, sp2d):
    f = pl.kernel(
        _seg_body,
        out_type=(jax.ShapeDtypeStruct((_NCORES, _NSPP, _NF), jnp.float32),
                  jax.ShapeDtypeStruct((_NW, _NSPP), jnp.float32)),
        mesh=plsc.VectorSubcoreMesh(core_axis_name="c", subcore_axis_name="s"),
        compiler_params=_SC_PARAMS,
        scratch_types=[
            pltpu.VMEM((_SBCH, _CHUNK), jnp.int32),
            pltpu.VMEM((2, _CHUNK, _NF), jnp.float32),
            pltpu.VMEM((_NSPP,), jnp.float32),
            pltpu.VMEM_SHARED((_NSPP, _NF), jnp.float32),
            pltpu.SemaphoreType.DMA,
            pltpu.SemaphoreType.DMA,
        ],
    )
    return f(feat, sp2d)


def _proj_body(ps_ref, pc_ref, wn_ref, we_ref, npot_ref, tab_ref):
    sums = ps_ref[0] + ps_ref[1]
    cnts = jnp.sum(pc_ref[...], axis=0)
    inv = 1.0 / jnp.clip(cnts, 1e-12, None)
    nf = sums * inv[:, None]

    def dot_t(a, b):
        return lax.dot_general(a, b, (((1,), (1,)), ((), ())),
                               preferred_element_type=jnp.float32,
                               precision=lax.Precision.HIGHEST)

    npot_ref[...] = dot_t(nf, wn_ref[...])
    zpad = jnp.zeros((64 - _NCLS, _NF), jnp.float32)
    wab = jnp.concatenate([we_ref[:, :_NF], zpad, we_ref[:, _NF:], zpad], axis=0)
    tab_ref[...] = dot_t(nf, wab)


_BLK = 1280


def _phase2(psums, pcnts, W_node, W_edge):
    nblk = _NSPP // _BLK
    return pl.pallas_call(
        _proj_body,
        grid=(nblk,),
        in_specs=[
            pl.BlockSpec((2, _BLK, _NF), lambda i: (0, i, 0)),
            pl.BlockSpec((_NW, _BLK), lambda i: (0, i)),
            pl.BlockSpec((_NCLS, _NF), lambda i: (0, 0)),
            pl.BlockSpec((_NCLS, 2 * _NF), lambda i: (0, 0)),
        ],
        out_specs=[
            pl.BlockSpec((_BLK, _NCLS), lambda i: (i, 0)),
            pl.BlockSpec((_BLK, _NF), lambda i: (i, 0)),
        ],
        out_shape=(jax.ShapeDtypeStruct((_NSP, _NCLS), jnp.float32),
                   jax.ShapeDtypeStruct((_NSPP, _NF), jnp.float32)),
    )(psums, pcnts, W_node, W_edge)


def _edge_body(tab_hbm, i0_hbm, i1_hbm, out_hbm,
               i0_v, i1_v, ba_v, bb_v, out_v, tab_sh, sga, sgb):
    cid = lax.axis_index("c")
    sid = lax.axis_index("s")
    wid = cid * _NSUB + sid

    seg = _NSPP // _NSUB
    pltpu.sync_copy(tab_hbm.at[pl.ds(sid * seg, seg)],
                    tab_sh.at[pl.ds(sid * seg, seg)])
    plsc.subcore_barrier()
    ebase = wid * _EW

    @pl.when(wid < _NEW)
    def _():
        pltpu.sync_copy(i0_hbm.at[wid], i0_v)
        pltpu.sync_copy(i1_hbm.at[wid], i1_v)

        @pl.loop(0, _NECH)
        def _(j):
            ca = pltpu.async_copy(tab_sh.at[i0_v.at[j]], ba_v, sga)
            cb = pltpu.async_copy(tab_sh.at[i1_v.at[j]], bb_v, sgb)
            ca.wait()
            cb.wait()

            @pl.loop(0, _ECHUNK)
            def _(i):
                out_v[i, pl.ds(0, 16)] = ba_v[i, pl.ds(0, 16)] + bb_v[i, pl.ds(64, 16)]
                out_v[i, pl.ds(5, 16)] = ba_v[i, pl.ds(5, 16)] + bb_v[i, pl.ds(69, 16)]

            pltpu.sync_copy(out_v, out_hbm.at[pl.ds(ebase + j * _ECHUNK, _ECHUNK)])


def _phase3(tab, e0, e1):
    f = pl.kernel(
        _edge_body,
        out_type=jax.ShapeDtypeStruct((_NE, _NCLS), jnp.float32),
        mesh=plsc.VectorSubcoreMesh(core_axis_name="c", subcore_axis_name="s"),
        compiler_params=_SC_PARAMS,
        scratch_types=[
            pltpu.VMEM((_NECH, _ECHUNK), jnp.int32),
            pltpu.VMEM((_NECH, _ECHUNK), jnp.int32),
            pltpu.VMEM((_ECHUNK, _NF), jnp.float32),
            pltpu.VMEM((_ECHUNK, _NF), jnp.float32),
            pltpu.VMEM((_ECHUNK, _NCLS), jnp.float32),
            pltpu.VMEM_SHARED((_NSPP, _NF), jnp.float32),
            pltpu.SemaphoreType.DMA,
            pltpu.SemaphoreType.DMA,
        ],
    )
    return f(tab, e0, e1)


def kernel(image, superpixel, edge_indexes, W_node, W_edge):
    feat = image.reshape(_NPIX, _NF)
    sp2d = superpixel.reshape(_NPIX // _CHUNK, _CHUNK)
    e0 = jnp.reshape(edge_indexes[:, 0], (_NEW, _NECH, _ECHUNK))
    e1 = jnp.reshape(edge_indexes[:, 1], (_NEW, _NECH, _ECHUNK))

    psums, pcnts = _phase1(feat, sp2d)
    npot, tab = _phase2(psums, pcnts, W_node, W_edge)
    ep = _phase3(tab, e0, e1)
    return (npot, ep)
